# Initial kernel scaffold; baseline (speedup 1.0000x reference)
import jax
import jax.numpy as jnp
from jax.experimental import pallas as pl


def kernel(rgb_mask, colors):
    raise NotImplementedError("write your pallas kernel here")


# speedup vs baseline: 44.6637x; 44.6637x over previous
import jax
import jax.numpy as jnp
from jax.experimental import pallas as pl
from jax.experimental.pallas import tpu as pltpu

_K = 64
_LANES = 128
_BH = 128


def _bf16_rne(x):
    xi = jax.lax.bitcast_convert_type(x, jnp.int32)
    r = (xi + 0x7FFF + ((xi >> 16) & 1)) & jnp.int32(-65536)
    return jax.lax.bitcast_convert_type(r, jnp.float32)


def _tc_body(cn_ref, x_ref, o_ref):
    r0 = x_ref[0, 0]
    g0 = x_ref[0, 1]
    b0 = x_ref[0, 2]
    nrm = jnp.sqrt(r0 * r0 + g0 * g0 + b0 * b0)
    r = _bf16_rne(r0 / nrm)
    g = _bf16_rne(g0 / nrm)
    b = _bf16_rne(b0 / nrm)
    best_s = jnp.full(r.shape, -1.0, jnp.float32)
    best_r = jnp.zeros(r.shape, jnp.float32)
    best_g = jnp.zeros(r.shape, jnp.float32)
    best_b = jnp.zeros(r.shape, jnp.float32)
    for k in range(_K):
        s = r * cn_ref[k, 0] + g * cn_ref[k, 1] + b * cn_ref[k, 2]
        m = s > best_s
        best_s = jnp.where(m, s, best_s)
        best_r = jnp.where(m, cn_ref[k, 3], best_r)
        best_g = jnp.where(m, cn_ref[k, 4], best_g)
        best_b = jnp.where(m, cn_ref[k, 5], best_b)
    nz = (r0 + g0 + b0) > 0.0
    zero = jnp.zeros(r.shape, jnp.float32)
    o_ref[0, 0] = jnp.where(nz, best_r, zero)
    o_ref[0, 1] = jnp.where(nz, best_g, zero)
    o_ref[0, 2] = jnp.where(nz, best_b, zero)


def kernel(rgb_mask, colors):
    B, C, H, W = rgb_mask.shape
    hw = H * W
    rows = hw // _LANES
    x = rgb_mask.reshape(B, C, rows, _LANES)
    a_norm = jnp.linalg.norm(colors, ord=2, axis=-1)
    cn = colors / a_norm[:, None]
    cnr = _bf16_rne(cn)
    grid = (B, rows // _BH)
    out = pl.pallas_call(
        _tc_body,
        grid=grid,
        in_specs=[
            pl.BlockSpec(memory_space=pltpu.SMEM),
            pl.BlockSpec((1, C, _BH, _LANES), lambda i, j: (i, 0, j, 0)),
        ],
        out_specs=pl.BlockSpec((1, C, _BH, _LANES), lambda i, j: (i, 0, j, 0)),
        out_shape=jax.ShapeDtypeStruct((B, C, rows, _LANES), jnp.float32),
    )(jnp.concatenate([cnr, cn], axis=1), x)
    return out.reshape(B, C, H, W)
